# initial kernel scaffold (unmeasured)
import jax
import jax.numpy as jnp
from jax import lax
from jax.experimental import pallas as pl
from jax.experimental.pallas import tpu as pltpu


def kernel(
    x,
):
    def body(*refs):
        pass

    out_shape = jax.ShapeDtypeStruct(..., jnp.float32)
    return pl.pallas_call(body, out_shape=out_shape)(...)


# baseline (device time: 13802 ns/iter reference)
import jax
import jax.numpy as jnp
from jax import lax
from jax.experimental import pallas as pl
from jax.experimental.pallas import tpu as pltpu

N_DEV = 8


def kernel(x):
    _, m, n = x.shape

    def body(x_ref, out_ref, send_buf, comm_ref, send_sems, recv_sems):
        my = lax.axis_index("i")

        send_buf[...] = x_ref[0].astype(jnp.bfloat16)

        barrier_sem = pltpu.get_barrier_semaphore()
        for off in range(1, N_DEV):
            pl.semaphore_signal(
                barrier_sem, inc=1,
                device_id=(lax.rem(my + off, N_DEV),),
                device_id_type=pl.DeviceIdType.MESH,
            )
        pl.semaphore_wait(barrier_sem, N_DEV - 1)

        rdmas = []
        for off in range(1, N_DEV):
            dst = lax.rem(my + off, N_DEV)
            rdma = pltpu.make_async_remote_copy(
                src_ref=send_buf,
                dst_ref=comm_ref.at[off - 1],
                send_sem=send_sems.at[off - 1],
                recv_sem=recv_sems.at[off - 1],
                device_id=(dst,),
                device_id_type=pl.DeviceIdType.MESH,
            )
            rdma.start()
            rdmas.append(rdma)

        for rdma in rdmas:
            rdma.wait()

        acc = x_ref[0]
        for k in range(N_DEV - 1):
            acc = acc + comm_ref[k].astype(jnp.float32)
        out_ref[...] = acc

    return pl.pallas_call(
        body,
        out_shape=jax.ShapeDtypeStruct((m, n), jnp.float32),
        in_specs=[pl.BlockSpec(memory_space=pltpu.VMEM)],
        out_specs=pl.BlockSpec(memory_space=pltpu.VMEM),
        scratch_shapes=[
            pltpu.VMEM((m, n), jnp.bfloat16),
            pltpu.VMEM((N_DEV - 1, m, n), jnp.bfloat16),
            pltpu.SemaphoreType.DMA((N_DEV - 1,)),
            pltpu.SemaphoreType.DMA((N_DEV - 1,)),
        ],
        compiler_params=pltpu.CompilerParams(collective_id=0),
    )(x)


# device time: 13106 ns/iter; 1.0531x vs baseline; 1.0531x over previous
import jax
import jax.numpy as jnp
from jax import lax
from jax.experimental import pallas as pl
from jax.experimental.pallas import tpu as pltpu

N_DEV = 8


def kernel(x):
    _, m, n = x.shape
    rows = m // N_DEV

    def body(x_ref, out_ref, stage, rs_ref, bcast, ag_ref,
             p1_send, p1_recv, p2_send, p2_recv):
        my = lax.axis_index("i")

        stage[...] = x_ref[0].astype(jnp.bfloat16)

        barrier_sem = pltpu.get_barrier_semaphore()
        for off in range(1, N_DEV):
            pl.semaphore_signal(
                barrier_sem, inc=1,
                device_id=(lax.rem(my + off, N_DEV),),
                device_id_type=pl.DeviceIdType.MESH,
            )
        pl.semaphore_wait(barrier_sem, N_DEV - 1)

        p1 = []
        for off in range(1, N_DEV):
            dst = lax.rem(my + off, N_DEV)
            rdma = pltpu.make_async_remote_copy(
                src_ref=stage.at[pl.ds(dst * rows, rows)],
                dst_ref=rs_ref.at[off - 1],
                send_sem=p1_send.at[off - 1],
                recv_sem=p1_recv.at[off - 1],
                device_id=(dst,),
                device_id_type=pl.DeviceIdType.MESH,
            )
            rdma.start()
            p1.append(rdma)

        for rdma in p1:
            rdma.wait()

        acc = x_ref[0, pl.ds(my * rows, rows)]
        for k in range(N_DEV - 1):
            acc = acc + rs_ref[k].astype(jnp.float32)
        out_ref[pl.ds(my * rows, rows), :] = acc
        bcast[...] = acc.astype(jnp.bfloat16)

        p2 = []
        for off in range(1, N_DEV):
            dst = lax.rem(my + off, N_DEV)
            rdma = pltpu.make_async_remote_copy(
                src_ref=bcast,
                dst_ref=ag_ref.at[off - 1],
                send_sem=p2_send.at[off - 1],
                recv_sem=p2_recv.at[off - 1],
                device_id=(dst,),
                device_id_type=pl.DeviceIdType.MESH,
            )
            rdma.start()
            p2.append(rdma)

        for off in range(1, N_DEV):
            p2[off - 1].wait()
            src = lax.rem(my - off + N_DEV, N_DEV)
            out_ref[pl.ds(src * rows, rows), :] = (
                ag_ref[off - 1].astype(jnp.float32)
            )

    return pl.pallas_call(
        body,
        out_shape=jax.ShapeDtypeStruct((m, n), jnp.float32),
        in_specs=[pl.BlockSpec(memory_space=pltpu.VMEM)],
        out_specs=pl.BlockSpec(memory_space=pltpu.VMEM),
        scratch_shapes=[
            pltpu.VMEM((m, n), jnp.bfloat16),
            pltpu.VMEM((N_DEV - 1, rows, n), jnp.bfloat16),
            pltpu.VMEM((rows, n), jnp.bfloat16),
            pltpu.VMEM((N_DEV - 1, rows, n), jnp.bfloat16),
            pltpu.SemaphoreType.DMA((N_DEV - 1,)),
            pltpu.SemaphoreType.DMA((N_DEV - 1,)),
            pltpu.SemaphoreType.DMA((N_DEV - 1,)),
            pltpu.SemaphoreType.DMA((N_DEV - 1,)),
        ],
        compiler_params=pltpu.CompilerParams(collective_id=0),
    )(x)
